# Initial kernel scaffold; baseline (speedup 1.0000x reference)
import jax
import jax.numpy as jnp
from jax.experimental import pallas as pl


def kernel(input, target):
    raise NotImplementedError("write your pallas kernel here")


# speedup vs baseline: 307.6454x; 307.6454x over previous
import jax
import jax.numpy as jnp
import numpy as np
from jax.experimental import pallas as pl
from jax.experimental.pallas import tpu as pltpu

_EPSILON = 2.5
_BIG = np.float32(1e9)


def _edt_mean_kernel(inp_ref, tgt_ref, out_ref, buf_a, buf_b, sc_ref, *, nb):
    b = pl.program_id(0)
    S = tgt_ref.shape[1]
    shp = (S, S, S)

    @pl.when(b == 0)
    def _init():
        sc_ref[0] = 0.0
        sc_ref[1] = 0.0

    t = tgt_ref[0]
    has_true = jnp.any(t > 0.0)
    buf_a[...] = jnp.where(t > 0.0, 0.0, _BIG)

    io0 = jax.lax.broadcasted_iota(jnp.int32, shp, 0).astype(jnp.float32)
    io1 = jax.lax.broadcasted_iota(jnp.int32, shp, 1).astype(jnp.float32)

    buf_b[...] = jnp.full(shp, _BIG * 4)

    def body0(k, _):
        kf = k.astype(jnp.float32)
        row = buf_a[pl.ds(k, 1), :, :]
        buf_b[...] = jnp.minimum(buf_b[...], row + (io0 - kf) ** 2)
        return 0

    jax.lax.fori_loop(0, S, body0, 0)

    buf_a[...] = jnp.full(shp, _BIG * 4)

    def body1(k, _):
        kf = k.astype(jnp.float32)
        row = buf_b[:, pl.ds(k, 1), :]
        buf_a[...] = jnp.minimum(buf_a[...], row + (io1 - kf) ** 2)
        return 0

    jax.lax.fori_loop(0, S, body1, 0)

    iox = jax.lax.broadcasted_iota(jnp.int32, shp, 2)
    buf_b[...] = buf_a[...]

    def body2(d, _):
        df = d.astype(jnp.float32)
        c = df * df
        v = buf_a[...]
        vp = pltpu.roll(v, d, 2)
        vp = jnp.where(iox >= d, vp, _BIG * 4)
        vm = pltpu.roll(v, S - d, 2)
        vm = jnp.where(iox < S - d, vm, _BIG * 4)
        buf_b[...] = jnp.minimum(buf_b[...], jnp.minimum(vp, vm) + c)
        return 0

    jax.lax.fori_loop(1, S, body2, 0)

    dist = jnp.sqrt(buf_b[...])
    dist = jnp.where(has_true, dist, 0.0)

    pm = inp_ref[0] > _EPSILON
    sc_ref[0] += jnp.sum(jnp.where(pm, dist, 0.0))
    sc_ref[1] += jnp.sum(pm.astype(jnp.float32))

    @pl.when(b == nb - 1)
    def _fin():
        tot = sc_ref[0]
        cnt = sc_ref[1]
        out_ref[0] = jnp.where(cnt > 0.0, tot / cnt, 0.0)


def kernel(input, target):
    B = int(np.prod(input.shape[:-3])) if input.ndim > 3 else 1
    D, H, W = input.shape[-3:]
    assert D == H == W, "kernel assumes a cubic lattice"
    inp = input.reshape(B, D, H, W).astype(jnp.float32)
    tgt = target.reshape(B, D, H, W).astype(jnp.float32)

    import functools

    out = pl.pallas_call(
        functools.partial(_edt_mean_kernel, nb=B),
        grid=(B,),
        in_specs=[
            pl.BlockSpec((1, D, H, W), lambda b: (b, 0, 0, 0)),
            pl.BlockSpec((1, D, H, W), lambda b: (b, 0, 0, 0)),
        ],
        out_specs=pl.BlockSpec(memory_space=pltpu.SMEM),
        out_shape=jax.ShapeDtypeStruct((1,), jnp.float32),
        scratch_shapes=[
            pltpu.VMEM((D, H, W), jnp.float32),
            pltpu.VMEM((D, H, W), jnp.float32),
            pltpu.SMEM((2,), jnp.float32),
        ],
        compiler_params=pltpu.CompilerParams(
            dimension_semantics=("arbitrary",),
        ),
    )(inp, tgt)
    return out[0]


# speedup vs baseline: 529.7922x; 1.7221x over previous
import jax
import jax.numpy as jnp
import numpy as np
from jax.experimental import pallas as pl
from jax.experimental.pallas import tpu as pltpu

_EPSILON = 2.5
_BIG = np.float32(1e9)


def _edt_mean_kernel(inp_ref, tgt_ref, out_ref, buf_a, buf_b, *, s):
    S = s
    shp = tgt_ref.shape
    L = shp[2]

    t = tgt_ref[...]
    buf_a[...] = jnp.where(t > 0.0, 0.0, _BIG)

    io0 = jax.lax.broadcasted_iota(jnp.int32, shp, 0).astype(jnp.float32)
    io1 = jax.lax.broadcasted_iota(jnp.int32, shp, 1).astype(jnp.float32)

    buf_b[...] = jnp.full(shp, _BIG * 4)

    def body0(k, _):
        kf = k.astype(jnp.float32)
        row = buf_a[pl.ds(k, 1), :, :]
        buf_b[...] = jnp.minimum(buf_b[...], row + (io0 - kf) ** 2)
        return 0

    jax.lax.fori_loop(0, S, body0, 0)

    buf_a[...] = jnp.full(shp, _BIG * 4)

    def body1(k, _):
        kf = k.astype(jnp.float32)
        row = buf_b[:, pl.ds(k, 1), :]
        buf_a[...] = jnp.minimum(buf_a[...], row + (io1 - kf) ** 2)
        return 0

    jax.lax.fori_loop(0, S, body1, 0)

    iox = jax.lax.broadcasted_iota(jnp.int32, shp, 2) % S
    buf_b[...] = buf_a[...]

    def body2(d, _):
        df = d.astype(jnp.float32)
        c = df * df
        v = buf_a[...]
        vp = pltpu.roll(v, d, 2)
        vp = jnp.where(iox >= d, vp, _BIG * 4)
        vm = pltpu.roll(v, L - d, 2)
        vm = jnp.where(iox < S - d, vm, _BIG * 4)
        buf_b[...] = jnp.minimum(buf_b[...], jnp.minimum(vp, vm) + c)
        return 0

    jax.lax.fori_loop(1, S, body2, 0)

    d2 = buf_b[...]
    dist = jnp.where(d2 >= 1e8, 0.0, jnp.sqrt(d2))

    pm = inp_ref[...] > _EPSILON
    tot = jnp.sum(jnp.where(pm, dist, 0.0))
    cnt = jnp.sum(pm.astype(jnp.float32))
    out_ref[0] = jnp.where(cnt > 0.0, tot / cnt, 0.0)


def kernel(input, target):
    B = int(np.prod(input.shape[:-3])) if input.ndim > 3 else 1
    D, H, W = input.shape[-3:]
    assert D == H == W, "kernel assumes a cubic lattice"
    S = W
    inp = jnp.transpose(
        input.reshape(B, D, H, W).astype(jnp.float32), (1, 2, 0, 3)
    ).reshape(D, H, B * S)
    tgt = jnp.transpose(
        target.reshape(B, D, H, W).astype(jnp.float32), (1, 2, 0, 3)
    ).reshape(D, H, B * S)

    import functools

    out = pl.pallas_call(
        functools.partial(_edt_mean_kernel, s=S),
        out_specs=pl.BlockSpec(memory_space=pltpu.SMEM),
        out_shape=jax.ShapeDtypeStruct((1,), jnp.float32),
        scratch_shapes=[
            pltpu.VMEM((D, H, B * S), jnp.float32),
            pltpu.VMEM((D, H, B * S), jnp.float32),
        ],
    )(inp, tgt)
    return out[0]


# speedup vs baseline: 551.7331x; 1.0414x over previous
import functools

import jax
import jax.numpy as jnp
import numpy as np
from jax.experimental import pallas as pl
from jax.experimental.pallas import tpu as pltpu

_EPSILON = 2.5
_BIG = np.float32(1e9)


def _edt_mean_kernel(inp_ref, tgt_ref, out_ref, buf_a, buf_b, tp_ref, tm_ref, *, s):
    S = s
    shp = tgt_ref.shape
    L = shp[2]

    t = tgt_ref[...]
    buf_a[...] = jnp.where(t > 0.0, 0.0, _BIG)

    iod = jax.lax.broadcasted_iota(jnp.int32, (S, 1, L), 0)
    ioxm = jax.lax.broadcasted_iota(jnp.int32, (S, 1, L), 2) % S
    iodf = iod.astype(jnp.float32)
    tp_ref[...] = jnp.where(ioxm >= iod, iodf * iodf, _BIG * 4)
    tm_ref[...] = jnp.where(ioxm < S - iod, iodf * iodf, _BIG * 4)

    io0 = jax.lax.broadcasted_iota(jnp.int32, (S, 1, 1), 0).astype(jnp.float32)
    io1 = jax.lax.broadcasted_iota(jnp.int32, (1, S, 1), 1).astype(jnp.float32)

    buf_b[...] = jnp.full(shp, _BIG * 4)

    def body0(k2, _):
        k = 2 * k2
        kf = k.astype(jnp.float32)
        r0 = buf_a[pl.ds(k, 1), :, :]
        r1 = buf_a[pl.ds(k + 1, 1), :, :]
        acc = jnp.minimum(buf_b[...], r0 + (io0 - kf) ** 2)
        acc = jnp.minimum(acc, r1 + (io0 - (kf + 1.0)) ** 2)
        buf_b[...] = acc
        return 0

    jax.lax.fori_loop(0, S // 2, body0, 0)

    buf_a[...] = jnp.full(shp, _BIG * 4)

    def body1(k2, _):
        k = 2 * k2
        kf = k.astype(jnp.float32)
        r0 = buf_b[:, pl.ds(k, 1), :]
        r1 = buf_b[:, pl.ds(k + 1, 1), :]
        acc = jnp.minimum(buf_a[...], r0 + (io1 - kf) ** 2)
        acc = jnp.minimum(acc, r1 + (io1 - (kf + 1.0)) ** 2)
        buf_a[...] = acc
        return 0

    jax.lax.fori_loop(0, S // 2, body1, 0)

    buf_b[...] = buf_a[...]

    def body2(d, _):
        v = buf_a[...]
        vp = pltpu.roll(v, d, 2) + tp_ref[pl.ds(d, 1), :, :]
        vm = pltpu.roll(v, L - d, 2) + tm_ref[pl.ds(d, 1), :, :]
        buf_b[...] = jnp.minimum(buf_b[...], jnp.minimum(vp, vm))
        return 0

    jax.lax.fori_loop(1, S, body2, 0)

    d2 = buf_b[...]
    dist = jnp.where(d2 >= 1e8, 0.0, jnp.sqrt(d2))

    pm = inp_ref[...] > _EPSILON
    tot = jnp.sum(jnp.where(pm, dist, 0.0))
    cnt = jnp.sum(pm.astype(jnp.float32))
    out_ref[0] = jnp.where(cnt > 0.0, tot / cnt, 0.0)


def kernel(input, target):
    B = int(np.prod(input.shape[:-3])) if input.ndim > 3 else 1
    D, H, W = input.shape[-3:]
    assert D == H == W, "kernel assumes a cubic lattice"
    S = W
    inp = jnp.transpose(
        input.reshape(B, D, H, W).astype(jnp.float32), (1, 2, 0, 3)
    ).reshape(D, H, B * S)
    tgt = jnp.transpose(
        target.reshape(B, D, H, W).astype(jnp.float32), (1, 2, 0, 3)
    ).reshape(D, H, B * S)

    out = pl.pallas_call(
        functools.partial(_edt_mean_kernel, s=S),
        out_specs=pl.BlockSpec(memory_space=pltpu.SMEM),
        out_shape=jax.ShapeDtypeStruct((1,), jnp.float32),
        scratch_shapes=[
            pltpu.VMEM((D, H, B * S), jnp.float32),
            pltpu.VMEM((D, H, B * S), jnp.float32),
            pltpu.VMEM((S, 1, B * S), jnp.float32),
            pltpu.VMEM((S, 1, B * S), jnp.float32),
        ],
    )(inp, tgt)
    return out[0]


# speedup vs baseline: 2086.8633x; 3.7824x over previous
import functools

import jax
import jax.numpy as jnp
import numpy as np
from jax.experimental import pallas as pl
from jax.experimental.pallas import tpu as pltpu

_EPSILON = 2.5
_BIG = np.float32(1e9)


def _edt_mean_kernel(inp_ref, tgt_ref, out_ref, buf_a, buf_b, *, s):
    S = s
    shp = tgt_ref.shape
    L = shp[2]

    t = tgt_ref[...]
    f = jnp.where(t > 0.0, 0.0, _BIG)
    buf_a[...] = f
    buf_b[...] = f

    ioxl = jax.lax.broadcasted_iota(jnp.int32, (1, 1, L), 2) % S
    j = 1
    while j < S:
        a = buf_a[...]
        r = pltpu.roll(a, j, 2)
        buf_a[...] = jnp.minimum(a, jnp.where(ioxl >= j, r, _BIG) + jnp.float32(j))
        b = buf_b[...]
        r = pltpu.roll(b, L - j, 2)
        buf_b[...] = jnp.minimum(b, jnp.where(ioxl < S - j, r, _BIG) + jnp.float32(j))
        j *= 2

    dx = jnp.minimum(buf_a[...], buf_b[...])
    buf_a[...] = dx * dx

    io0 = jax.lax.broadcasted_iota(jnp.int32, (S, 1, 1), 0).astype(jnp.float32)
    io1 = jax.lax.broadcasted_iota(jnp.int32, (1, S, 1), 1).astype(jnp.float32)

    buf_b[...] = jnp.full(shp, _BIG * _BIG)

    def body1(k4, _):
        k = 4 * k4
        kf = k.astype(jnp.float32)
        acc = buf_b[...]
        for i in range(4):
            row = buf_a[:, pl.ds(k + i, 1), :]
            acc = jnp.minimum(acc, row + (io1 - (kf + i)) ** 2)
        buf_b[...] = acc
        return 0

    jax.lax.fori_loop(0, S // 4, body1, 0)

    buf_a[...] = jnp.full(shp, _BIG * _BIG)

    def body0(k4, _):
        k = 4 * k4
        kf = k.astype(jnp.float32)
        acc = buf_a[...]
        for i in range(4):
            row = buf_b[pl.ds(k + i, 1), :, :]
            acc = jnp.minimum(acc, row + (io0 - (kf + i)) ** 2)
        buf_a[...] = acc
        return 0

    jax.lax.fori_loop(0, S // 4, body0, 0)

    d2 = buf_a[...]
    dist = jnp.where(d2 >= 1e8, 0.0, jnp.sqrt(d2))

    pm = inp_ref[...] > _EPSILON
    tot = jnp.sum(jnp.where(pm, dist, 0.0))
    cnt = jnp.sum(pm.astype(jnp.float32))
    out_ref[0] = jnp.where(cnt > 0.0, tot / cnt, 0.0)


def kernel(input, target):
    B = int(np.prod(input.shape[:-3])) if input.ndim > 3 else 1
    D, H, W = input.shape[-3:]
    assert D == H == W, "kernel assumes a cubic lattice"
    S = W
    inp = jnp.transpose(
        input.reshape(B, D, H, W).astype(jnp.float32), (1, 2, 0, 3)
    ).reshape(D, H, B * S)
    tgt = jnp.transpose(
        target.reshape(B, D, H, W).astype(jnp.float32), (1, 2, 0, 3)
    ).reshape(D, H, B * S)

    out = pl.pallas_call(
        functools.partial(_edt_mean_kernel, s=S),
        out_specs=pl.BlockSpec(memory_space=pltpu.SMEM),
        out_shape=jax.ShapeDtypeStruct((1,), jnp.float32),
        scratch_shapes=[
            pltpu.VMEM((D, H, B * S), jnp.float32),
            pltpu.VMEM((D, H, B * S), jnp.float32),
        ],
    )(inp, tgt)
    return out[0]
